# baseline (speedup 1.0000x reference)
import functools

import jax
import jax.numpy as jnp
from jax import lax
from jax.experimental import pallas as pl
from jax.experimental.pallas import tpu as pltpu
from jax.experimental.pallas import tpu_sc as plsc


def _sc_info():
    try:
        info = plsc.get_sparse_core_info()
        return info.num_cores, info.num_subcores
    except Exception:
        return 2, 16


@functools.partial(jax.jit, static_argnames=("nw",))
def _linearize_table(tableT, tail, *, nw):
    d, V = tableT.shape
    W = 128
    nfull = V // W
    Vpad = (V + W - 1) // W * W
    ntail = V - nfull * W
    nmain = nfull // nw * nw
    nextra = nfull - nmain
    per_w = nmain // nw
    out_rows = Vpad // 2
    mesh = plsc.VectorSubcoreMesh(core_axis_name="c", subcore_axis_name="s")

    nbuf = 2

    @functools.partial(
        pl.kernel,
        out_type=jax.ShapeDtypeStruct((out_rows, 2 * d), jnp.float32),
        mesh=mesh,
        compiler_params=pltpu.CompilerParams(
            use_tc_tiling_on_sc=True, needs_layout_passes=False
        ),
        scratch_types=[
            pltpu.VMEM((nbuf, d, W), jnp.float32),
            pltpu.VMEM((nbuf, W // 2, 2 * d), jnp.float32),
            pltpu.SemaphoreType.DMA,
            pltpu.SemaphoreType.DMA,
        ],
    )
    def k(tabT_hbm, tail_hbm, out_hbm, in_v, out_v, in_sem, out_sem):
        c = lax.axis_index("c")
        s = lax.axis_index("s")
        w = s * 2 + c
        t0 = w * per_w

        iota = lax.iota(jnp.int32, 16)
        row_idx = [iota + (16 * kk) % d for kk in range(8)]

        def transpose_block(b, npairs):
            for p in range(npairs):
                for kk in range(8):
                    vrel = 2 * p + (16 * kk) // d
                    col_idx = jnp.full((16,), vrel, jnp.int32)
                    vals = plsc.load_gather(in_v.at[b], [row_idx[kk], col_idx])
                    out_v[b, p, pl.ds(16 * kk, 16)] = vals

        for b in range(nbuf):
            pltpu.async_copy(
                tabT_hbm.at[:, pl.ds((t0 + b) * W, W)], in_v.at[b], in_sem
            )

        def body(g, carry):
            for b in range(nbuf):
                t = nbuf * g + b
                tv = t0 + t
                pltpu.make_async_copy(
                    tabT_hbm.at[:, pl.ds(tv * W, W)], in_v.at[b], in_sem
                ).wait()

                @pl.when(t >= nbuf)
                def _():
                    pltpu.make_async_copy(
                        out_v.at[b], out_hbm.at[pl.ds(0, W // 2)], out_sem
                    ).wait()

                transpose_block(b, W // 2)
                pltpu.async_copy(
                    out_v.at[b], out_hbm.at[pl.ds(tv * (W // 2), W // 2)], out_sem
                )

                @pl.when(t + nbuf < per_w)
                def _():
                    pltpu.async_copy(
                        tabT_hbm.at[:, pl.ds((tv + nbuf) * W, W)],
                        in_v.at[b],
                        in_sem,
                    )
            return carry

        lax.fori_loop(0, per_w // nbuf, body, 0)
        for _ in range(nbuf):
            pltpu.make_async_copy(
                out_v.at[0], out_hbm.at[pl.ds(0, W // 2)], out_sem
            ).wait()

        @pl.when(w < nextra)
        def _():
            tv = nmain + w
            pltpu.sync_copy(tabT_hbm.at[:, pl.ds(tv * W, W)], in_v.at[0])
            transpose_block(0, W // 2)
            pltpu.sync_copy(out_v.at[0], out_hbm.at[pl.ds(tv * (W // 2), W // 2)])

        @pl.when(w == nw - 1)
        def _():
            pltpu.sync_copy(tail_hbm, in_v.at[0])
            transpose_block(0, W // 2)
            pltpu.sync_copy(
                out_v.at[0], out_hbm.at[pl.ds(nfull * (W // 2), W // 2)]
            )

    return k(tableT, tail)


@functools.partial(jax.jit, static_argnames=("nw", "steps", "seq", "d"))
def _gather_pool(idx3, table, *, nw, steps, seq, d):
    ipg = 2 * seq
    pairs_total = nw * steps
    mesh = plsc.VectorSubcoreMesh(core_axis_name="c", subcore_axis_name="s")

    nbuf = 4
    nflush = 2
    steps_per_flush = steps // nflush
    assert steps_per_flush % nbuf == 0

    @functools.partial(
        pl.kernel,
        out_type=(
            jax.ShapeDtypeStruct((pairs_total, d), jnp.float32),
            jax.ShapeDtypeStruct((pairs_total, d), jnp.float32),
        ),
        mesh=mesh,
        compiler_params=pltpu.CompilerParams(use_tc_tiling_on_sc=False),
        scratch_types=[
            pltpu.VMEM((steps, ipg), jnp.int32),
            pltpu.VMEM((nbuf, ipg, d), jnp.float32),
            pltpu.VMEM((steps_per_flush, d), jnp.float32),
            pltpu.VMEM((steps_per_flush, d), jnp.float32),
            pltpu.SemaphoreType.DMA,
        ],
    )
    def k(idx_hbm, table_hbm, out1_hbm, out2_hbm, idx_v, rows_v, s1_v, s2_v, sem):
        c = lax.axis_index("c")
        s = lax.axis_index("s")
        w = s * 2 + c

        pltpu.sync_copy(idx_hbm.at[w], idx_v)
        for b in range(nbuf):
            pltpu.async_copy(table_hbm.at[idx_v.at[b]], rows_v.at[b], sem)

        for h in range(nflush):

            def body(g, carry, h=h):
                for b in range(nbuf):
                    jloc = nbuf * g + b
                    j = h * steps_per_flush + jloc
                    pltpu.make_async_copy(
                        table_hbm.at[idx_v.at[j]], rows_v.at[b], sem
                    ).wait()
                    for snt, slab in ((0, s1_v), (1, s2_v)):
                        srow = seq * snt
                        for kk in range(d // 16):
                            col = pl.ds(16 * kk, 16)
                            accs = [rows_v[b, srow + i, col] for i in range(4)]
                            for base in range(4, seq, 4):
                                for i in range(4):
                                    if base + i < seq:
                                        accs[i] = accs[i] + rows_v[b, srow + base + i, col]
                            slab[jloc, col] = (accs[0] + accs[1]) + (accs[2] + accs[3])

                    @pl.when(j + nbuf < steps)
                    def _():
                        pltpu.async_copy(
                            table_hbm.at[idx_v.at[j + nbuf]], rows_v.at[b], sem
                        )
                return carry

            lax.fori_loop(0, steps_per_flush // nbuf, body, 0)
            out_off = w * steps + h * steps_per_flush
            pltpu.sync_copy(s1_v, out1_hbm.at[pl.ds(out_off, steps_per_flush)])
            pltpu.sync_copy(s2_v, out2_hbm.at[pl.ds(out_off, steps_per_flush)])

    return k(idx3, table)


@functools.partial(jax.jit, static_argnames=("seq",))
def _project_cosine(pooled1, pooled2, Wt, b2, *, seq):
    B, d = pooled1.shape
    p = Wt.shape[1]
    blk = 1024
    inv = 1.0 / float(seq)

    def body(s1_ref, s2_ref, wt_ref, b_ref, out_ref):
        wt = wt_ref[...]
        bb = b_ref[...]
        s1 = s1_ref[...] * inv
        s2 = s2_ref[...] * inv
        p1 = jnp.dot(s1, wt, preferred_element_type=jnp.float32) + bb
        p2 = jnp.dot(s2, wt, preferred_element_type=jnp.float32) + bb
        d12 = jnp.sum(p1 * p2, axis=1)
        n1 = jnp.maximum(jnp.sqrt(jnp.sum(p1 * p1, axis=1)), 1e-8)
        n2 = jnp.maximum(jnp.sqrt(jnp.sum(p2 * p2, axis=1)), 1e-8)
        out_ref[...] = (d12 / (n1 * n2)).reshape(blk, 1)

    out = pl.pallas_call(
        body,
        grid=(B // blk,),
        in_specs=[
            pl.BlockSpec((blk, d), lambda i: (i, 0)),
            pl.BlockSpec((blk, d), lambda i: (i, 0)),
            pl.BlockSpec((d, p), lambda i: (0, 0)),
            pl.BlockSpec((1, p), lambda i: (0, 0)),
        ],
        out_specs=pl.BlockSpec((blk, 1), lambda i: (i, 0)),
        out_shape=jax.ShapeDtypeStruct((B, 1), jnp.float32),
    )(pooled1, pooled2, Wt, b2)
    return out.reshape(B)


def kernel(x, table, W, b):
    B, two, seq = x.shape
    assert two == 2
    d = table.shape[1]
    nc, ns = _sc_info()
    nw = nc * ns

    steps = B // nw
    assert steps * nw == B
    assert d == 64

    idx3 = x.astype(jnp.int32).reshape(nw, steps, 2 * seq)

    V = table.shape[0]
    VW = 128
    nfull = V // VW
    ntail = V - nfull * VW
    assert ntail > 0 and ntail % 2 == 0
    Vpad = (nfull + 1) * VW
    tableT = table.T
    tail = jnp.pad(table[V - ntail :, :].T, ((0, 0), (0, VW - ntail)))
    tableL = _linearize_table(tableT, tail, nw=nw)
    tableLin = tableL.reshape(Vpad, d)

    pooled1, pooled2 = _gather_pool(idx3, tableLin, nw=nw, steps=steps, seq=seq, d=d)

    Wt = W.T
    b2 = b.reshape(1, -1)
    return _project_cosine(pooled1, pooled2, Wt, b2, seq=seq)


# speedup vs baseline: 2.0327x; 2.0327x over previous
import functools

import jax
import jax.numpy as jnp
from jax import lax
from jax.experimental import pallas as pl
from jax.experimental.pallas import tpu as pltpu
from jax.experimental.pallas import tpu_sc as plsc


def _sc_info():
    try:
        info = plsc.get_sparse_core_info()
        return info.num_cores, info.num_subcores
    except Exception:
        return 2, 16


@jax.jit
def _linearize_table(tableT):
    d, V = tableT.shape
    BV = 2048
    nblk = -(-V // BV)
    Vpad = (V + 127) // 128 * 128
    out_rows = Vpad // 2

    def body(in_ref, out_ref):
        at = in_ref[...].T
        at3 = at.reshape(BV // 2, 2, d)
        out_ref[...] = jnp.concatenate([at3[:, 0, :], at3[:, 1, :]], axis=1)

    return pl.pallas_call(
        body,
        grid=(nblk,),
        in_specs=[pl.BlockSpec((d, BV), lambda i: (0, i))],
        out_specs=pl.BlockSpec((BV // 2, 2 * d), lambda i: (i, 0)),
        out_shape=jax.ShapeDtypeStruct((out_rows, 2 * d), jnp.float32),
    )(tableT)


@functools.partial(jax.jit, static_argnames=("nw", "steps", "seq", "d"))
def _gather_pool(idx3, table, *, nw, steps, seq, d):
    ipg = 2 * seq
    pairs_total = nw * steps
    mesh = plsc.VectorSubcoreMesh(core_axis_name="c", subcore_axis_name="s")

    nbuf = 4
    nflush = 2
    steps_per_flush = steps // nflush
    assert steps_per_flush % nbuf == 0

    @functools.partial(
        pl.kernel,
        out_type=(
            jax.ShapeDtypeStruct((pairs_total, d), jnp.float32),
            jax.ShapeDtypeStruct((pairs_total, d), jnp.float32),
        ),
        mesh=mesh,
        compiler_params=pltpu.CompilerParams(use_tc_tiling_on_sc=False),
        scratch_types=[
            pltpu.VMEM((steps, ipg), jnp.int32),
            pltpu.VMEM((nbuf, ipg, d), jnp.float32),
            pltpu.VMEM((steps_per_flush, d), jnp.float32),
            pltpu.VMEM((steps_per_flush, d), jnp.float32),
            pltpu.SemaphoreType.DMA,
        ],
    )
    def k(idx_hbm, table_hbm, out1_hbm, out2_hbm, idx_v, rows_v, s1_v, s2_v, sem):
        c = lax.axis_index("c")
        s = lax.axis_index("s")
        w = s * 2 + c

        pltpu.sync_copy(idx_hbm.at[w], idx_v)
        for b in range(nbuf):
            pltpu.async_copy(table_hbm.at[idx_v.at[b]], rows_v.at[b], sem)

        for h in range(nflush):

            def body(g, carry, h=h):
                for b in range(nbuf):
                    jloc = nbuf * g + b
                    j = h * steps_per_flush + jloc
                    pltpu.make_async_copy(
                        table_hbm.at[idx_v.at[j]], rows_v.at[b], sem
                    ).wait()
                    for snt, slab in ((0, s1_v), (1, s2_v)):
                        srow = seq * snt
                        for kk in range(d // 16):
                            col = pl.ds(16 * kk, 16)
                            accs = [rows_v[b, srow + i, col] for i in range(4)]
                            for base in range(4, seq, 4):
                                for i in range(4):
                                    if base + i < seq:
                                        accs[i] = accs[i] + rows_v[b, srow + base + i, col]
                            slab[jloc, col] = (accs[0] + accs[1]) + (accs[2] + accs[3])

                    @pl.when(j + nbuf < steps)
                    def _():
                        pltpu.async_copy(
                            table_hbm.at[idx_v.at[j + nbuf]], rows_v.at[b], sem
                        )
                return carry

            lax.fori_loop(0, steps_per_flush // nbuf, body, 0)
            out_off = w * steps + h * steps_per_flush
            pltpu.sync_copy(s1_v, out1_hbm.at[pl.ds(out_off, steps_per_flush)])
            pltpu.sync_copy(s2_v, out2_hbm.at[pl.ds(out_off, steps_per_flush)])

    return k(idx3, table)


@functools.partial(jax.jit, static_argnames=("seq",))
def _project_cosine(pooled1, pooled2, Wt, b2, *, seq):
    B, d = pooled1.shape
    p = Wt.shape[1]
    blk = 1024
    inv = 1.0 / float(seq)

    def body(s1_ref, s2_ref, wt_ref, b_ref, out_ref):
        wt = wt_ref[...]
        bb = b_ref[...]
        s1 = s1_ref[...] * inv
        s2 = s2_ref[...] * inv
        p1 = jnp.dot(s1, wt, preferred_element_type=jnp.float32) + bb
        p2 = jnp.dot(s2, wt, preferred_element_type=jnp.float32) + bb
        d12 = jnp.sum(p1 * p2, axis=1)
        n1 = jnp.maximum(jnp.sqrt(jnp.sum(p1 * p1, axis=1)), 1e-8)
        n2 = jnp.maximum(jnp.sqrt(jnp.sum(p2 * p2, axis=1)), 1e-8)
        out_ref[...] = (d12 / (n1 * n2)).reshape(blk, 1)

    out = pl.pallas_call(
        body,
        grid=(B // blk,),
        in_specs=[
            pl.BlockSpec((blk, d), lambda i: (i, 0)),
            pl.BlockSpec((blk, d), lambda i: (i, 0)),
            pl.BlockSpec((d, p), lambda i: (0, 0)),
            pl.BlockSpec((1, p), lambda i: (0, 0)),
        ],
        out_specs=pl.BlockSpec((blk, 1), lambda i: (i, 0)),
        out_shape=jax.ShapeDtypeStruct((B, 1), jnp.float32),
    )(pooled1, pooled2, Wt, b2)
    return out.reshape(B)


def kernel(x, table, W, b):
    B, two, seq = x.shape
    assert two == 2
    d = table.shape[1]
    nc, ns = _sc_info()
    nw = nc * ns

    steps = B // nw
    assert steps * nw == B
    assert d == 64

    idx3 = x.astype(jnp.int32).reshape(nw, steps, 2 * seq)

    V = table.shape[0]
    VW = 128
    nfull = V // VW
    ntail = V - nfull * VW
    assert ntail > 0 and ntail % 2 == 0
    Vpad = (nfull + 1) * VW
    tableT = table.T
    tableL = _linearize_table(tableT)
    tableLin = tableL.reshape(Vpad, d)

    pooled1, pooled2 = _gather_pool(idx3, tableLin, nw=nw, steps=steps, seq=seq, d=d)

    Wt = W.T
    b2 = b.reshape(1, -1)
    return _project_cosine(pooled1, pooled2, Wt, b2, seq=seq)


# speedup vs baseline: 2.3600x; 1.1610x over previous
import functools

import jax
import jax.numpy as jnp
from jax import lax
from jax.experimental import pallas as pl
from jax.experimental.pallas import tpu as pltpu
from jax.experimental.pallas import tpu_sc as plsc


def _sc_info():
    try:
        info = plsc.get_sparse_core_info()
        return info.num_cores, info.num_subcores
    except Exception:
        return 2, 16


@jax.jit
def _linearize_table(tableT):
    d, V = tableT.shape
    BV = 8192
    nblk = -(-V // BV)
    Vpad = (V + 127) // 128 * 128
    out_rows = Vpad // 2

    def body(in_ref, out_ref):
        at = in_ref[...].T
        at3 = at.reshape(BV // 2, 2, d)
        out_ref[...] = jnp.concatenate([at3[:, 0, :], at3[:, 1, :]], axis=1)

    return pl.pallas_call(
        body,
        grid=(nblk,),
        in_specs=[pl.BlockSpec((d, BV), lambda i: (0, i))],
        out_specs=pl.BlockSpec((BV // 2, 2 * d), lambda i: (i, 0)),
        out_shape=jax.ShapeDtypeStruct((out_rows, 2 * d), jnp.float32),
    )(tableT)


@functools.partial(jax.jit, static_argnames=("nw", "steps", "seq", "d"))
def _gather_pool(idx3, table, *, nw, steps, seq, d):
    ipg = 2 * seq
    pairs_total = nw * steps
    mesh = plsc.VectorSubcoreMesh(core_axis_name="c", subcore_axis_name="s")

    nbuf = 4
    nflush = 2
    steps_per_flush = steps // nflush
    assert steps_per_flush % nbuf == 0

    @functools.partial(
        pl.kernel,
        out_type=(
            jax.ShapeDtypeStruct((pairs_total, d), jnp.float32),
            jax.ShapeDtypeStruct((pairs_total, d), jnp.float32),
        ),
        mesh=mesh,
        compiler_params=pltpu.CompilerParams(use_tc_tiling_on_sc=False),
        scratch_types=[
            pltpu.VMEM((steps, ipg), jnp.int32),
            pltpu.VMEM((nbuf, ipg, d), jnp.float32),
            pltpu.VMEM((steps_per_flush, d), jnp.float32),
            pltpu.VMEM((steps_per_flush, d), jnp.float32),
            pltpu.SemaphoreType.DMA,
        ],
    )
    def k(idx_hbm, table_hbm, out1_hbm, out2_hbm, idx_v, rows_v, s1_v, s2_v, sem):
        c = lax.axis_index("c")
        s = lax.axis_index("s")
        w = s * 2 + c

        pltpu.sync_copy(idx_hbm.at[w], idx_v)
        for b in range(nbuf):
            pltpu.async_copy(table_hbm.at[idx_v.at[b]], rows_v.at[b], sem)

        for h in range(nflush):

            def body(g, carry, h=h):
                for b in range(nbuf):
                    jloc = nbuf * g + b
                    j = h * steps_per_flush + jloc
                    pltpu.make_async_copy(
                        table_hbm.at[idx_v.at[j]], rows_v.at[b], sem
                    ).wait()
                    for snt, slab in ((0, s1_v), (1, s2_v)):
                        srow = seq * snt
                        for kk in range(d // 16):
                            col = pl.ds(16 * kk, 16)
                            accs = [rows_v[b, srow + i, col] for i in range(4)]
                            for base in range(4, seq, 4):
                                for i in range(4):
                                    if base + i < seq:
                                        accs[i] = accs[i] + rows_v[b, srow + base + i, col]
                            slab[jloc, col] = (accs[0] + accs[1]) + (accs[2] + accs[3])

                    @pl.when(j + nbuf < steps)
                    def _():
                        pltpu.async_copy(
                            table_hbm.at[idx_v.at[j + nbuf]], rows_v.at[b], sem
                        )
                return carry

            lax.fori_loop(0, steps_per_flush // nbuf, body, 0)
            out_off = w * steps + h * steps_per_flush
            pltpu.sync_copy(s1_v, out1_hbm.at[pl.ds(out_off, steps_per_flush)])
            pltpu.sync_copy(s2_v, out2_hbm.at[pl.ds(out_off, steps_per_flush)])

    return k(idx3, table)


@functools.partial(jax.jit, static_argnames=("seq",))
def _project_cosine(pooled1, pooled2, Wt, b2, *, seq):
    B, d = pooled1.shape
    p = Wt.shape[1]
    blk = 1024
    inv = 1.0 / float(seq)

    def body(s1_ref, s2_ref, wt_ref, b_ref, out_ref):
        wt = wt_ref[...]
        bb = b_ref[...]
        s1 = s1_ref[...] * inv
        s2 = s2_ref[...] * inv
        p1 = jnp.dot(s1, wt, preferred_element_type=jnp.float32) + bb
        p2 = jnp.dot(s2, wt, preferred_element_type=jnp.float32) + bb
        d12 = jnp.sum(p1 * p2, axis=1)
        n1 = jnp.maximum(jnp.sqrt(jnp.sum(p1 * p1, axis=1)), 1e-8)
        n2 = jnp.maximum(jnp.sqrt(jnp.sum(p2 * p2, axis=1)), 1e-8)
        out_ref[...] = (d12 / (n1 * n2)).reshape(blk, 1)

    out = pl.pallas_call(
        body,
        grid=(B // blk,),
        in_specs=[
            pl.BlockSpec((blk, d), lambda i: (i, 0)),
            pl.BlockSpec((blk, d), lambda i: (i, 0)),
            pl.BlockSpec((d, p), lambda i: (0, 0)),
            pl.BlockSpec((1, p), lambda i: (0, 0)),
        ],
        out_specs=pl.BlockSpec((blk, 1), lambda i: (i, 0)),
        out_shape=jax.ShapeDtypeStruct((B, 1), jnp.float32),
    )(pooled1, pooled2, Wt, b2)
    return out.reshape(B)


def kernel(x, table, W, b):
    B, two, seq = x.shape
    assert two == 2
    d = table.shape[1]
    nc, ns = _sc_info()
    nw = nc * ns

    steps = B // nw
    assert steps * nw == B
    assert d == 64

    idx3 = x.astype(jnp.int32).reshape(nw, steps, 2 * seq)

    V = table.shape[0]
    VW = 128
    nfull = V // VW
    ntail = V - nfull * VW
    assert ntail > 0 and ntail % 2 == 0
    Vpad = (nfull + 1) * VW
    tableT = table.T
    tableL = _linearize_table(tableT)
    tableLin = tableL.reshape(Vpad, d)

    pooled1, pooled2 = _gather_pool(idx3, tableLin, nw=nw, steps=steps, seq=seq, d=d)

    Wt = W.T
    b2 = b.reshape(1, -1)
    return _project_cosine(pooled1, pooled2, Wt, b2, seq=seq)


# speedup vs baseline: 2.3730x; 1.0055x over previous
import functools

import jax
import jax.numpy as jnp
from jax import lax
from jax.experimental import pallas as pl
from jax.experimental.pallas import tpu as pltpu
from jax.experimental.pallas import tpu_sc as plsc


def _sc_info():
    try:
        info = plsc.get_sparse_core_info()
        return info.num_cores, info.num_subcores
    except Exception:
        return 2, 16


@jax.jit
def _linearize_table(tableT):
    d, V = tableT.shape
    BV = 16384
    nblk = -(-V // BV)
    Vpad = (V + 127) // 128 * 128
    out_rows = Vpad // 2

    def body(in_ref, out_ref):
        at = in_ref[...].T
        at3 = at.reshape(BV // 2, 2, d)
        out_ref[...] = jnp.concatenate([at3[:, 0, :], at3[:, 1, :]], axis=1)

    return pl.pallas_call(
        body,
        grid=(nblk,),
        in_specs=[pl.BlockSpec((d, BV), lambda i: (0, i))],
        out_specs=pl.BlockSpec((BV // 2, 2 * d), lambda i: (i, 0)),
        out_shape=jax.ShapeDtypeStruct((out_rows, 2 * d), jnp.float32),
    )(tableT)


@functools.partial(jax.jit, static_argnames=("nw", "steps", "seq", "d"))
def _gather_pool(idx3, table, *, nw, steps, seq, d):
    ipg = 2 * seq
    pairs_total = nw * steps
    mesh = plsc.VectorSubcoreMesh(core_axis_name="c", subcore_axis_name="s")

    nbuf = 4
    nflush = 2
    steps_per_flush = steps // nflush
    assert steps_per_flush % nbuf == 0

    @functools.partial(
        pl.kernel,
        out_type=(
            jax.ShapeDtypeStruct((pairs_total, d), jnp.float32),
            jax.ShapeDtypeStruct((pairs_total, d), jnp.float32),
        ),
        mesh=mesh,
        compiler_params=pltpu.CompilerParams(use_tc_tiling_on_sc=False),
        scratch_types=[
            pltpu.VMEM((steps, ipg), jnp.int32),
            pltpu.VMEM((nbuf, ipg, d), jnp.float32),
            pltpu.VMEM((steps_per_flush, d), jnp.float32),
            pltpu.VMEM((steps_per_flush, d), jnp.float32),
            pltpu.SemaphoreType.DMA,
        ],
    )
    def k(idx_hbm, table_hbm, out1_hbm, out2_hbm, idx_v, rows_v, s1_v, s2_v, sem):
        c = lax.axis_index("c")
        s = lax.axis_index("s")
        w = s * 2 + c

        pltpu.sync_copy(idx_hbm.at[w], idx_v)
        for b in range(nbuf):
            pltpu.async_copy(table_hbm.at[idx_v.at[b]], rows_v.at[b], sem)

        for h in range(nflush):

            def body(g, carry, h=h):
                for b in range(nbuf):
                    jloc = nbuf * g + b
                    j = h * steps_per_flush + jloc
                    pltpu.make_async_copy(
                        table_hbm.at[idx_v.at[j]], rows_v.at[b], sem
                    ).wait()
                    for snt, slab in ((0, s1_v), (1, s2_v)):
                        srow = seq * snt
                        for kk in range(d // 16):
                            col = pl.ds(16 * kk, 16)
                            accs = [rows_v[b, srow + i, col] for i in range(4)]
                            for base in range(4, seq, 4):
                                for i in range(4):
                                    if base + i < seq:
                                        accs[i] = accs[i] + rows_v[b, srow + base + i, col]
                            slab[jloc, col] = (accs[0] + accs[1]) + (accs[2] + accs[3])

                    @pl.when(j + nbuf < steps)
                    def _():
                        pltpu.async_copy(
                            table_hbm.at[idx_v.at[j + nbuf]], rows_v.at[b], sem
                        )
                return carry

            lax.fori_loop(0, steps_per_flush // nbuf, body, 0)
            out_off = w * steps + h * steps_per_flush
            pltpu.sync_copy(s1_v, out1_hbm.at[pl.ds(out_off, steps_per_flush)])
            pltpu.sync_copy(s2_v, out2_hbm.at[pl.ds(out_off, steps_per_flush)])

    return k(idx3, table)


@functools.partial(jax.jit, static_argnames=("seq",))
def _project_cosine(pooled1, pooled2, Wt, b2, *, seq):
    B, d = pooled1.shape
    p = Wt.shape[1]
    blk = 1024
    inv = 1.0 / float(seq)

    def body(s1_ref, s2_ref, wt_ref, b_ref, out_ref):
        wt = wt_ref[...]
        bb = b_ref[...]
        s1 = s1_ref[...] * inv
        s2 = s2_ref[...] * inv
        p1 = jnp.dot(s1, wt, preferred_element_type=jnp.float32) + bb
        p2 = jnp.dot(s2, wt, preferred_element_type=jnp.float32) + bb
        d12 = jnp.sum(p1 * p2, axis=1)
        n1 = jnp.maximum(jnp.sqrt(jnp.sum(p1 * p1, axis=1)), 1e-8)
        n2 = jnp.maximum(jnp.sqrt(jnp.sum(p2 * p2, axis=1)), 1e-8)
        out_ref[...] = (d12 / (n1 * n2)).reshape(blk, 1)

    out = pl.pallas_call(
        body,
        grid=(B // blk,),
        in_specs=[
            pl.BlockSpec((blk, d), lambda i: (i, 0)),
            pl.BlockSpec((blk, d), lambda i: (i, 0)),
            pl.BlockSpec((d, p), lambda i: (0, 0)),
            pl.BlockSpec((1, p), lambda i: (0, 0)),
        ],
        out_specs=pl.BlockSpec((blk, 1), lambda i: (i, 0)),
        out_shape=jax.ShapeDtypeStruct((B, 1), jnp.float32),
    )(pooled1, pooled2, Wt, b2)
    return out.reshape(B)


def kernel(x, table, W, b):
    B, two, seq = x.shape
    assert two == 2
    d = table.shape[1]
    nc, ns = _sc_info()
    nw = nc * ns

    steps = B // nw
    assert steps * nw == B
    assert d == 64

    idx3 = x.astype(jnp.int32).reshape(nw, steps, 2 * seq)

    V = table.shape[0]
    VW = 128
    nfull = V // VW
    ntail = V - nfull * VW
    assert ntail > 0 and ntail % 2 == 0
    Vpad = (nfull + 1) * VW
    tableT = table.T
    tableL = _linearize_table(tableT)
    tableLin = tableL.reshape(Vpad, d)

    pooled1, pooled2 = _gather_pool(idx3, tableLin, nw=nw, steps=steps, seq=seq, d=d)

    Wt = W.T
    b2 = b.reshape(1, -1)
    return _project_cosine(pooled1, pooled2, Wt, b2, seq=seq)


# speedup vs baseline: 2.4006x; 1.0116x over previous
import functools

import jax
import jax.numpy as jnp
from jax import lax
from jax.experimental import pallas as pl
from jax.experimental.pallas import tpu as pltpu
from jax.experimental.pallas import tpu_sc as plsc


def _sc_info():
    try:
        info = plsc.get_sparse_core_info()
        return info.num_cores, info.num_subcores
    except Exception:
        return 2, 16


@jax.jit
def _linearize_table(tableT):
    d, V = tableT.shape
    BV = 16384
    nblk = -(-V // BV)
    Vpad = (V + 127) // 128 * 128
    out_rows = Vpad // 2

    def body(in_ref, out_ref):
        at = in_ref[...].T
        at3 = at.reshape(BV // 2, 2, d)
        out_ref[...] = jnp.concatenate([at3[:, 0, :], at3[:, 1, :]], axis=1)

    return pl.pallas_call(
        body,
        grid=(nblk,),
        in_specs=[pl.BlockSpec((d, BV), lambda i: (0, i))],
        out_specs=pl.BlockSpec((BV // 2, 2 * d), lambda i: (i, 0)),
        out_shape=jax.ShapeDtypeStruct((out_rows, 2 * d), jnp.float32),
    )(tableT)


@functools.partial(jax.jit, static_argnames=("nw", "steps", "seq", "d"))
def _gather_pool(idx3, table, *, nw, steps, seq, d):
    ipg = 2 * seq
    pairs_total = nw * steps
    mesh = plsc.VectorSubcoreMesh(core_axis_name="c", subcore_axis_name="s")

    nbuf = 4
    nflush = 2
    steps_per_flush = steps // nflush
    assert steps_per_flush % nbuf == 0

    @functools.partial(
        pl.kernel,
        out_type=(
            jax.ShapeDtypeStruct((pairs_total, d), jnp.float32),
            jax.ShapeDtypeStruct((pairs_total, d), jnp.float32),
        ),
        mesh=mesh,
        compiler_params=pltpu.CompilerParams(use_tc_tiling_on_sc=False),
        scratch_types=[
            pltpu.VMEM((steps, ipg), jnp.int32),
            pltpu.VMEM((nbuf, ipg, d), jnp.float32),
            pltpu.VMEM((steps_per_flush, d), jnp.float32),
            pltpu.VMEM((steps_per_flush, d), jnp.float32),
            pltpu.SemaphoreType.DMA,
        ],
    )
    def k(idx_hbm, table_hbm, out1_hbm, out2_hbm, idx_v, rows_v, s1_v, s2_v, sem):
        c = lax.axis_index("c")
        s = lax.axis_index("s")
        w = s * 2 + c

        pltpu.sync_copy(idx_hbm.at[w], idx_v)
        for b in range(nbuf):
            pltpu.async_copy(table_hbm.at[idx_v.at[b]], rows_v.at[b], sem)

        for h in range(nflush):

            def body(g, carry, h=h):
                for b in range(nbuf):
                    jloc = nbuf * g + b
                    j = h * steps_per_flush + jloc
                    pltpu.make_async_copy(
                        table_hbm.at[idx_v.at[j]], rows_v.at[b], sem
                    ).wait()
                    for snt, slab in ((0, s1_v), (1, s2_v)):
                        srow = seq * snt
                        for kk in range(d // 16):
                            col = pl.ds(16 * kk, 16)
                            accs = [rows_v[b, srow + i, col] for i in range(4)]
                            for base in range(4, seq, 4):
                                for i in range(4):
                                    if base + i < seq:
                                        accs[i] = accs[i] + rows_v[b, srow + base + i, col]
                            slab[jloc, col] = (accs[0] + accs[1]) + (accs[2] + accs[3])

                    @pl.when(j + nbuf < steps)
                    def _():
                        pltpu.async_copy(
                            table_hbm.at[idx_v.at[j + nbuf]], rows_v.at[b], sem
                        )
                return carry

            lax.fori_loop(0, steps_per_flush // nbuf, body, 0)
            out_off = w * steps + h * steps_per_flush
            pltpu.sync_copy(s1_v, out1_hbm.at[pl.ds(out_off, steps_per_flush)])
            pltpu.sync_copy(s2_v, out2_hbm.at[pl.ds(out_off, steps_per_flush)])

    return k(idx3, table)


@functools.partial(jax.jit, static_argnames=("seq",))
def _project_cosine(pooled1, pooled2, Wt, b2, *, seq):
    B, d = pooled1.shape
    p = Wt.shape[1]
    blk = 1024
    inv = 1.0 / float(seq)

    def body(s1_ref, s2_ref, wt_ref, b_ref, out_ref):
        wt = wt_ref[...]
        bb = b_ref[...]
        s1 = s1_ref[...] * inv
        s2 = s2_ref[...] * inv
        p1 = jnp.dot(s1, wt, preferred_element_type=jnp.float32) + bb
        p2 = jnp.dot(s2, wt, preferred_element_type=jnp.float32) + bb
        d12 = jnp.sum(p1 * p2, axis=1)
        n1 = jnp.maximum(jnp.sqrt(jnp.sum(p1 * p1, axis=1)), 1e-8)
        n2 = jnp.maximum(jnp.sqrt(jnp.sum(p2 * p2, axis=1)), 1e-8)
        out_ref[...] = (d12 / (n1 * n2)).reshape(blk, 1)

    out = pl.pallas_call(
        body,
        grid=(B // blk,),
        in_specs=[
            pl.BlockSpec((blk, d), lambda i: (i, 0)),
            pl.BlockSpec((blk, d), lambda i: (i, 0)),
            pl.BlockSpec((d, p), lambda i: (0, 0)),
            pl.BlockSpec((1, p), lambda i: (0, 0)),
        ],
        out_specs=pl.BlockSpec((blk, 1), lambda i: (i, 0)),
        out_shape=jax.ShapeDtypeStruct((B, 1), jnp.float32),
    )(pooled1, pooled2, Wt, b2)
    return out.reshape(B)


def kernel(x, table, W, b):
    B, two, seq = x.shape
    assert two == 2
    d = table.shape[1]
    nc, ns = _sc_info()
    nw = nc * ns

    steps = B // nw
    assert steps * nw == B
    assert d == 64

    idx3 = x.astype(jnp.int32).reshape(nw, steps, 2 * seq)

    V = table.shape[0]
    Vpad = (V + 127) // 128 * 128
    tableL = _linearize_table(table.T)
    tableLin = tableL.reshape(Vpad, d)

    pooled1, pooled2 = _gather_pool(idx3, tableLin, nw=nw, steps=steps, seq=seq, d=d)

    Wt = W.T
    b2 = b.reshape(1, -1)
    return _project_cosine(pooled1, pooled2, Wt, b2, seq=seq)
